# Initial kernel scaffold; baseline (speedup 1.0000x reference)
import jax
import jax.numpy as jnp
from jax.experimental import pallas as pl


def kernel(x, edge_index, batch, w1_0, b1_0, gi_0, bi_0, w2_0, b2_0, go_0, bo_0, w1_1, b1_1, gi_1, bi_1, w2_1, b2_1, go_1, bo_1, w1_2, b1_2, gi_2, bi_2, w2_2, b2_2, go_2, bo_2):
    raise NotImplementedError("write your pallas kernel here")


# speedup vs baseline: 6.0793x; 6.0793x over previous
import functools

import jax
import jax.numpy as jnp
from jax import lax
from jax.experimental import pallas as pl
from jax.experimental.pallas import tpu as pltpu
from jax.experimental.pallas import tpu_sc as plsc

N = 10000
E = 320000
D = 128
BN_EPS = 1e-5

NC = 2
NS = 16
NW = NC * NS

CHUNK = 128
NCHUNK = E // CHUNK
TPT = (NCHUNK + NW - 1) // NW
RPT = 624
RPT_LAST = N - 15 * RPT


def _seg_sum_body(h_hbm, src_hbm, dst_hbm, zeros_hbm, out_hbm,
                  sidx, didx, rows, acc, sem):
    cid = lax.axis_index("c")
    sid = lax.axis_index("s")
    wid = cid * NS + sid

    base = sid * RPT

    @pl.when(sid < NS - 1)
    def _():
        pltpu.sync_copy(zeros_hbm.at[pl.ds(base, RPT)],
                        acc.at[pl.ds(base, RPT)])

    @pl.when(sid == NS - 1)
    def _():
        pltpu.sync_copy(zeros_hbm.at[pl.ds(base, RPT_LAST)],
                        acc.at[pl.ds(base, RPT_LAST)])

    plsc.subcore_barrier()

    def step(t, carry):
        c = t * NW + wid

        @pl.when(c < NCHUNK)
        def _():
            off = pl.multiple_of(c * CHUNK, CHUNK)
            pltpu.sync_copy(src_hbm.at[pl.ds(off, CHUNK)], sidx)
            pltpu.sync_copy(dst_hbm.at[pl.ds(off, CHUNK)], didx)
            pltpu.async_copy(h_hbm.at[sidx], rows, sem).wait()
            pltpu.sync_copy(rows, acc.at[didx], add=True)

        return carry

    lax.fori_loop(0, TPT, step, 0)
    plsc.subcore_barrier()

    @pl.when(sid < NS - 1)
    def _():
        pltpu.sync_copy(acc.at[pl.ds(base, RPT)],
                        out_hbm.at[pl.ds(cid * N + base, RPT)])

    @pl.when(sid == NS - 1)
    def _():
        pltpu.sync_copy(acc.at[pl.ds(base, RPT_LAST)],
                        out_hbm.at[pl.ds(cid * N + base, RPT_LAST)])


_seg_sum = pl.kernel(
    _seg_sum_body,
    out_type=jax.ShapeDtypeStruct((NC * N, D), jnp.float32),
    mesh=plsc.VectorSubcoreMesh(core_axis_name="c", subcore_axis_name="s"),
    scratch_types=[
        pltpu.VMEM((CHUNK,), jnp.int32),
        pltpu.VMEM((CHUNK,), jnp.int32),
        pltpu.VMEM((CHUNK, D), jnp.float32),
        pltpu.VMEM_SHARED((N, D), jnp.float32),
        pltpu.SemaphoreType.DMA,
    ],
)


def _bn(a, g, b):
    m = jnp.mean(a, axis=0)
    v = jnp.mean((a - m) * (a - m), axis=0)
    return (a - m) * lax.rsqrt(v + BN_EPS) * g + b


def _dense_body(h_ref, part_ref, w1_ref, b1_ref, gi_ref, bi_ref,
                w2_ref, b2_ref, go_ref, bo_ref, o_ref, *, relu_out):
    s = h_ref[...] + part_ref[:N] + part_ref[N:]
    a = jnp.dot(s, w1_ref[...], preferred_element_type=jnp.float32)
    a = a + b1_ref[...]
    a = jnp.maximum(_bn(a, gi_ref[...], bi_ref[...]), 0.0)
    o = jnp.dot(a, w2_ref[...], preferred_element_type=jnp.float32)
    o = o + b2_ref[...]
    o = _bn(o, go_ref[...], bo_ref[...])
    if relu_out:
        o = jnp.maximum(o, 0.0)
    o_ref[...] = o


def _dense(h, part, w1, b1, gi, bi, w2, b2, go, bo, relu_out):
    return pl.pallas_call(
        functools.partial(_dense_body, relu_out=relu_out),
        out_shape=jax.ShapeDtypeStruct((N, D), jnp.float32),
    )(h, part, w1, b1, gi, bi, w2, b2, go, bo)


def kernel(x, edge_index, batch,
           w1_0, b1_0, gi_0, bi_0, w2_0, b2_0, go_0, bo_0,
           w1_1, b1_1, gi_1, bi_1, w2_1, b2_1, go_1, bo_1,
           w1_2, b1_2, gi_2, bi_2, w2_2, b2_2, go_2, bo_2):
    src = edge_index[0]
    dst = edge_index[1]
    zeros = jnp.zeros((N, D), jnp.float32)

    params = [
        (w1_0, b1_0, gi_0, bi_0, w2_0, b2_0, go_0, bo_0),
        (w1_1, b1_1, gi_1, bi_1, w2_1, b2_1, go_1, bo_1),
        (w1_2, b1_2, gi_2, bi_2, w2_2, b2_2, go_2, bo_2),
    ]

    h = x
    for l in range(3):
        part = _seg_sum(h, src, dst, zeros)
        h = _dense(h, part, *params[l], relu_out=(l < 2))
    return h
